# Initial kernel scaffold; baseline (speedup 1.0000x reference)
import jax
import jax.numpy as jnp
from jax.experimental import pallas as pl


def kernel(positions, species, sigma_matrix, epsilon_matrix, alpha_matrix):
    raise NotImplementedError("write your pallas kernel here")


# speedup vs baseline: 3349.6576x; 3349.6576x over previous
import jax
import jax.numpy as jnp
from jax.experimental import pallas as pl

_N = 2048
_BLOCK = 256
_NSPEC = 4
_CUTOFF = 1.0


def _body(pos_ref, posT_ref, sc_ref, sr_ref, posf_ref, sig_ref, eps_ref,
          alp_ref, e_ref, f_ref):
    pid = pl.program_id(0)
    n = posT_ref.shape[1]
    blk = pos_ref.shape[0]

    px = posT_ref[0:1, :]
    py = posT_ref[1:2, :]
    pz = posT_ref[2:3, :]
    cx = pos_ref[:, 0:1]
    cy = pos_ref[:, 1:2]
    cz = pos_ref[:, 2:3]
    dx = px - cx
    dy = py - cy
    dz = pz - cz
    d2 = dx * dx + dy * dy + dz * dz
    r = jnp.sqrt(d2)

    oh_i = (sc_ref[...] == jax.lax.broadcasted_iota(
        jnp.int32, (blk, _NSPEC), 1)).astype(jnp.float32)
    oh_j = (sr_ref[...] == jax.lax.broadcasted_iota(
        jnp.int32, (_NSPEC, n), 0)).astype(jnp.float32)
    sig = jnp.dot(jnp.dot(oh_i, sig_ref[...], preferred_element_type=jnp.float32),
                  oh_j, preferred_element_type=jnp.float32)
    eps = jnp.dot(jnp.dot(oh_i, eps_ref[...], preferred_element_type=jnp.float32),
                  oh_j, preferred_element_type=jnp.float32)
    alp = jnp.dot(jnp.dot(oh_i, alp_ref[...], preferred_element_type=jnp.float32),
                  oh_j, preferred_element_type=jnp.float32)

    row = pid * blk + jax.lax.broadcasted_iota(jnp.int32, (blk, n), 0)
    col = jax.lax.broadcasted_iota(jnp.int32, (blk, n), 1)
    mask = (row != col) & (r < _CUTOFF) & (r < sig)

    rm = jnp.where(mask, r, 0.0)
    x = 1.0 - rm / sig
    p1 = jnp.exp((alp - 1.0) * jnp.log(x))
    energies = jnp.where(mask, eps / alp * (p1 * x), 0.0)
    fmag = jnp.where(mask, -(eps / sig) * p1, 0.0)
    g = fmag / jnp.where(mask, r, 1.0)

    s = jnp.sum(g, axis=1, keepdims=True)
    f_ref[...] = (jnp.dot(g, posf_ref[...], preferred_element_type=jnp.float32)
                  - s * pos_ref[...])

    @pl.when(pid == 0)
    def _():
        e_ref[...] = jnp.zeros((1, 1), jnp.float32)

    e_ref[...] += 0.5 * jnp.sum(energies, keepdims=True)


def kernel(positions, species, sigma_matrix, epsilon_matrix, alpha_matrix):
    n = positions.shape[0]
    posT = positions.T
    spec_col = species.reshape(n, 1).astype(jnp.int32)
    spec_row = species.reshape(1, n).astype(jnp.int32)
    grid = (n // _BLOCK,)
    e2d, forces = pl.pallas_call(
        _body,
        grid=grid,
        in_specs=[
            pl.BlockSpec((_BLOCK, 3), lambda i: (i, 0)),
            pl.BlockSpec((3, n), lambda i: (0, 0)),
            pl.BlockSpec((_BLOCK, 1), lambda i: (i, 0)),
            pl.BlockSpec((1, n), lambda i: (0, 0)),
            pl.BlockSpec((n, 3), lambda i: (0, 0)),
            pl.BlockSpec((_NSPEC, _NSPEC), lambda i: (0, 0)),
            pl.BlockSpec((_NSPEC, _NSPEC), lambda i: (0, 0)),
            pl.BlockSpec((_NSPEC, _NSPEC), lambda i: (0, 0)),
        ],
        out_specs=[
            pl.BlockSpec((1, 1), lambda i: (0, 0)),
            pl.BlockSpec((_BLOCK, 3), lambda i: (i, 0)),
        ],
        out_shape=[
            jax.ShapeDtypeStruct((1, 1), jnp.float32),
            jax.ShapeDtypeStruct((n, 3), jnp.float32),
        ],
    )(positions, posT, spec_col, spec_row, positions,
      sigma_matrix, epsilon_matrix, alpha_matrix)
    return e2d[0, 0], forces
